# baseline (speedup 1.0000x reference)
import functools

import jax
import jax.numpy as jnp
from jax import lax
from jax.experimental import pallas as pl
from jax.experimental.pallas import tpu as pltpu
from jax.experimental.pallas import tpu_sc as plsc

_MAX_LENGTH = 200


def _sc_broadcast(table, batch, seq, nrows, dim):
    vp, lanes = table.shape
    seq_pad = (seq + 15) // 16 * 16
    sd = seq * dim
    mesh = plsc.VectorSubcoreMesh(core_axis_name="c", subcore_axis_name="s")
    nw = 32
    rows_per_w = batch // nw

    @functools.partial(
        pl.kernel, mesh=mesh,
        out_type=jax.ShapeDtypeStruct((batch, sd), jnp.float32),
        scratch_types=[
            pltpu.VMEM((seq_pad,), jnp.int32),
            pltpu.VMEM((seq_pad, lanes), jnp.float32),
            pltpu.VMEM((1, sd), jnp.float32),
            pltpu.SemaphoreType.DMA,
            pltpu.SemaphoreType.DMA,
        ],
    )
    def k(table_hbm, out_hbm, idx_v, rows_v, row_v, gsem, wsem):
        wid = lax.axis_index("s") * 2 + lax.axis_index("c")

        for i in range(seq_pad // 16):
            base = lax.iota(jnp.int32, 16) + (i * 16)
            pos = jnp.minimum(jnp.maximum(base, _MAX_LENGTH), nrows - 1)
            idx_v[pl.ds(i * 16, 16)] = pos

        pltpu.async_copy(table_hbm.at[idx_v], rows_v, gsem).wait()

        for s in range(seq):
            for c in range(dim // 16):
                row_v[0, pl.ds(s * dim + c * 16, 16)] = (
                    rows_v[s, pl.ds(c * 16, 16)])

        base_row = wid * rows_per_w
        copies = [
            pltpu.make_async_copy(
                row_v, out_hbm.at[pl.ds(base_row + j, 1)], wsem)
            for j in range(rows_per_w)
        ]
        for cp in copies:
            cp.start()
        for cp in copies:
            cp.wait()

    return k(table)


def kernel(inputs, kernel):
    batch, seq = inputs.shape
    nrows, dim = kernel.shape
    vp = (nrows + 7) // 8 * 8
    lanes = max(dim, 128)
    table = jnp.zeros((vp, lanes), kernel.dtype).at[:nrows, :dim].set(kernel)

    out = _sc_broadcast(table, batch, seq, nrows, dim)
    return out.reshape(batch, seq, dim)


# speedup vs baseline: 1.9171x; 1.9171x over previous
import functools

import jax
import jax.numpy as jnp
from jax import lax
from jax.experimental import pallas as pl
from jax.experimental.pallas import tpu as pltpu
from jax.experimental.pallas import tpu_sc as plsc

_MAX_LENGTH = 200


def _sc_gather(table, seq, nrows):
    vp, dim = table.shape
    seq_pad = (seq + 15) // 16 * 16
    mesh = plsc.VectorSubcoreMesh(core_axis_name="c", subcore_axis_name="s")

    @functools.partial(
        pl.kernel, mesh=mesh,
        out_type=jax.ShapeDtypeStruct((seq, dim), jnp.float32),
        scratch_types=[
            pltpu.VMEM((seq_pad,), jnp.int32),
            pltpu.VMEM((seq_pad, dim), jnp.float32),
            pltpu.SemaphoreType.DMA,
        ],
    )
    def k(table_hbm, out_hbm, idx_v, rows_v, sem):
        wid = lax.axis_index("s") * 2 + lax.axis_index("c")

        @pl.when(wid == 0)
        def _():
            for i in range(seq_pad // 16):
                base = lax.iota(jnp.int32, 16) + (i * 16)
                pos = jnp.minimum(jnp.maximum(base, _MAX_LENGTH), nrows - 1)
                idx_v[pl.ds(i * 16, 16)] = pos
            pltpu.async_copy(table_hbm.at[idx_v], rows_v, sem).wait()
            pltpu.sync_copy(rows_v.at[pl.ds(0, seq)], out_hbm)

    return k(table)


def _bcast_body(row_ref, out_ref, row_block, sems, *, tb, nblk):
    row_block[...] = jnp.broadcast_to(row_ref[...], row_block.shape)
    for i in range(nblk):
        pltpu.make_async_copy(
            row_block, out_ref.at[pl.ds(i * tb, tb), :], sems.at[i]).start()
    for i in range(nblk):
        pltpu.make_async_copy(
            row_block, out_ref.at[pl.ds(i * tb, tb), :], sems.at[i]).wait()


def kernel(inputs, kernel):
    batch, seq = inputs.shape
    nrows, dim = kernel.shape
    vp = (nrows + 7) // 8 * 8
    lanes = max(dim, 128)
    table = jnp.zeros((vp, lanes), kernel.dtype).at[:nrows, :dim].set(kernel)

    gathered = _sc_gather(table, seq, nrows)[:, :dim]

    row = gathered.reshape(1, seq * dim)
    tb = 256
    nblk = batch // tb
    out = pl.pallas_call(
        functools.partial(_bcast_body, tb=tb, nblk=nblk),
        in_specs=[pl.BlockSpec(memory_space=pltpu.VMEM)],
        out_specs=pl.BlockSpec(memory_space=pl.ANY),
        out_shape=jax.ShapeDtypeStruct((batch, seq * dim), jnp.float32),
        scratch_shapes=[
            pltpu.VMEM((tb, seq * dim), jnp.float32),
            pltpu.SemaphoreType.DMA((nblk,)),
        ],
    )(row)
    return out.reshape(batch, seq, dim)
